# Initial kernel scaffold; baseline (speedup 1.0000x reference)
import jax
import jax.numpy as jnp
from jax.experimental import pallas as pl


def kernel(x):
    raise NotImplementedError("write your pallas kernel here")


# speedup vs baseline: 29.4019x; 29.4019x over previous
import functools

import jax
import jax.numpy as jnp
from jax import lax
from jax.experimental import pallas as pl
from jax.experimental.pallas import tpu as pltpu
from jax.experimental.pallas import tpu_sc as plsc

NUM_BINS = 2048
L = 16
NC = 2
NS = 16
NW = NC * NS
N = 2 * 4096 * 4096
PER_W = N // NW
CHUNK = 16384
N_CHUNKS = PER_W // CHUNK
VECS = CHUNK // L

_mesh = plsc.VectorSubcoreMesh(core_axis_name="c", subcore_axis_name="s")


@functools.partial(
    pl.kernel,
    out_type=jax.ShapeDtypeStruct((NW, L), jnp.float32),
    mesh=_mesh,
    scratch_types=[
        pltpu.VMEM((CHUNK,), jnp.float32),
        pltpu.VMEM((L,), jnp.float32),
    ],
)
def _maxabs_kernel(x_hbm, out_hbm, buf, maxbuf):
    wid = lax.axis_index("s") * NC + lax.axis_index("c")
    base = wid * PER_W

    def chunk_body(j, vmax):
        pltpu.sync_copy(x_hbm.at[pl.ds(base + j * CHUNK, CHUNK)], buf)

        def inner(i, vm):
            v = buf[pl.ds(i * L, L)]
            return jnp.maximum(vm, jnp.abs(v))

        return lax.fori_loop(0, VECS, inner, vmax)

    vmax = lax.fori_loop(0, N_CHUNKS, chunk_body, jnp.zeros((L,), jnp.float32))
    maxbuf[...] = vmax
    pltpu.sync_copy(maxbuf, out_hbm.at[wid])


@functools.partial(
    pl.kernel,
    out_type=jax.ShapeDtypeStruct((NW, NUM_BINS), jnp.float32),
    mesh=_mesh,
    scratch_types=[
        pltpu.VMEM((CHUNK,), jnp.float32),
        pltpu.VMEM((L * NUM_BINS,), jnp.float32),
        pltpu.VMEM((NUM_BINS,), jnp.float32),
        pltpu.VMEM((L,), jnp.float32),
    ],
    compiler_params=pltpu.CompilerParams(needs_layout_passes=False),
)
def _hist_kernel(x_hbm, scale_hbm, out_hbm, buf, hist, final, scalebuf):
    wid = lax.axis_index("s") * NC + lax.axis_index("c")
    base = wid * PER_W

    pltpu.sync_copy(scale_hbm, scalebuf)
    vscale = scalebuf[...]

    zeros = jnp.zeros((L,), jnp.float32)

    def zbody(i, _):
        hist[pl.ds(i * L, L)] = zeros
        return 0

    lax.fori_loop(0, (L * NUM_BINS) // L, zbody, 0)

    lane_off = lax.iota(jnp.int32, L) * NUM_BINS
    ones = jnp.ones((L,), jnp.float32)

    def chunk_body(j, _):
        pltpu.sync_copy(x_hbm.at[pl.ds(base + j * CHUNK, CHUNK)], buf)

        def inner(i, _):
            v = buf[pl.ds(i * L, L)]
            t = jnp.abs(v) * vscale
            idx = jnp.minimum(t.astype(jnp.int32), NUM_BINS - 1)
            plsc.addupdate_scatter(hist, [idx + lane_off], ones)
            return 0

        lax.fori_loop(0, VECS, inner, 0)
        return 0

    lax.fori_loop(0, N_CHUNKS, chunk_body, 0)

    def rbody(b, _):
        acc = hist[pl.ds(b * L, L)]
        for lane in range(1, L):
            acc = acc + hist[pl.ds(lane * NUM_BINS + b * L, L)]
        final[pl.ds(b * L, L)] = acc
        return 0

    lax.fori_loop(0, NUM_BINS // L, rbody, 0)
    pltpu.sync_copy(final, out_hbm.at[wid])


def kernel(x):
    xf = x.reshape(-1)
    part_max = _maxabs_kernel(xf)
    x_max = jnp.max(part_max)
    scale = jnp.where(x_max > 0.0, NUM_BINS / x_max, 0.0)
    scale_vec = jnp.full((L,), scale, jnp.float32)
    part_hist = _hist_kernel(xf, scale_vec)
    hist = jnp.sum(part_hist, axis=0)
    edges = jnp.linspace(0.0, x_max, NUM_BINS + 1).astype(jnp.float32)
    return (x, hist, edges)


# speedup vs baseline: 105.8124x; 3.5988x over previous
import functools

import jax
import jax.numpy as jnp
from jax import lax
from jax.experimental import pallas as pl
from jax.experimental.pallas import tpu as pltpu
from jax.experimental.pallas import tpu_sc as plsc

NUM_BINS = 2048
L = 16
NC = 2
NS = 16
NW = NC * NS
N = 2 * 4096 * 4096
PER_W = N // NW
CHUNK = 32768
N_CHUNKS = PER_W // CHUNK
VECS = CHUNK // L
GROUP = 8

_mesh = plsc.VectorSubcoreMesh(core_axis_name="c", subcore_axis_name="s")


def _double_buffered(x_hbm, base, buf0, buf1, sem0, sem1, process):
    bufs = (buf0, buf1)
    sems = (sem0, sem1)

    def copy(c, slot):
        return pltpu.make_async_copy(
            x_hbm.at[pl.ds(base + c * CHUNK, CHUNK)], bufs[slot], sems[slot]
        )

    copy(0, 0).start()

    def pair_body(p, _):
        c0 = p * 2
        copy(c0 + 1, 1).start()
        copy(c0, 0).wait()
        process(buf0)

        @pl.when(c0 + 2 < N_CHUNKS)
        def _():
            copy(c0 + 2, 0).start()

        copy(c0 + 1, 1).wait()
        process(buf1)
        return 0

    lax.fori_loop(0, N_CHUNKS // 2, pair_body, 0)


@functools.partial(
    pl.kernel,
    out_type=jax.ShapeDtypeStruct((NW, L), jnp.float32),
    mesh=_mesh,
    scratch_types=[
        pltpu.VMEM((CHUNK,), jnp.float32),
        pltpu.VMEM((CHUNK,), jnp.float32),
        pltpu.VMEM((L,), jnp.float32),
        pltpu.SemaphoreType.DMA,
        pltpu.SemaphoreType.DMA,
    ],
    compiler_params=pltpu.CompilerParams(needs_layout_passes=False),
)
def _maxabs_kernel(x_hbm, out_hbm, buf0, buf1, maxbuf, sem0, sem1):
    wid = lax.axis_index("s") * NC + lax.axis_index("c")
    base = wid * PER_W
    maxbuf[...] = jnp.zeros((L,), jnp.float32)

    def process(slot_ref):
        def inner(i, vm):
            o = i * (GROUP * L)
            m = None
            for g in range(GROUP):
                a = jnp.abs(slot_ref[pl.ds(o + g * L, L)])
                m = a if m is None else jnp.maximum(m, a)
            return jnp.maximum(vm, m)

        vmax = plsc.parallel_loop(0, VECS // GROUP, carry=maxbuf[...])(inner)
        maxbuf[...] = vmax

    _double_buffered(x_hbm, base, buf0, buf1, sem0, sem1, process)
    pltpu.sync_copy(maxbuf, out_hbm.at[wid])


@functools.partial(
    pl.kernel,
    out_type=jax.ShapeDtypeStruct((NW, NUM_BINS), jnp.float32),
    mesh=_mesh,
    scratch_types=[
        pltpu.VMEM((CHUNK,), jnp.float32),
        pltpu.VMEM((CHUNK,), jnp.float32),
        pltpu.VMEM((L * NUM_BINS,), jnp.float32),
        pltpu.VMEM((NUM_BINS,), jnp.float32),
        pltpu.VMEM((L,), jnp.float32),
        pltpu.SemaphoreType.DMA,
        pltpu.SemaphoreType.DMA,
    ],
    compiler_params=pltpu.CompilerParams(needs_layout_passes=False),
)
def _hist_kernel(x_hbm, scale_hbm, out_hbm, buf0, buf1, hist, final, scalebuf, sem0, sem1):
    wid = lax.axis_index("s") * NC + lax.axis_index("c")
    base = wid * PER_W

    pltpu.sync_copy(scale_hbm, scalebuf)
    vscale = scalebuf[...]

    zeros = jnp.zeros((L,), jnp.float32)

    @plsc.parallel_loop(0, (L * NUM_BINS) // L, unroll=8)
    def _(i):
        hist[pl.ds(i * L, L)] = zeros

    lane_off = lax.iota(jnp.int32, L) * NUM_BINS
    ones = jnp.ones((L,), jnp.float32)

    def process(slot_ref):
        @plsc.parallel_loop(0, VECS // GROUP, unroll=2)
        def _(i):
            o = i * (GROUP * L)
            for g in range(GROUP):
                v = slot_ref[pl.ds(o + g * L, L)]
                t = jnp.abs(v) * vscale
                idx = jnp.minimum(t.astype(jnp.int32), NUM_BINS - 1)
                plsc.addupdate_scatter(hist, [idx + lane_off], ones)

    _double_buffered(x_hbm, base, buf0, buf1, sem0, sem1, process)

    @plsc.parallel_loop(0, NUM_BINS // L, unroll=2)
    def _(b):
        acc = hist[pl.ds(b * L, L)]
        for lane in range(1, L):
            acc = acc + hist[pl.ds(lane * NUM_BINS + b * L, L)]
        final[pl.ds(b * L, L)] = acc

    pltpu.sync_copy(final, out_hbm.at[wid])


def kernel(x):
    xf = x.reshape(-1)
    part_max = _maxabs_kernel(xf)
    x_max = jnp.max(part_max)
    scale = jnp.where(x_max > 0.0, NUM_BINS / x_max, 0.0)
    scale_vec = jnp.full((L,), scale, jnp.float32)
    part_hist = _hist_kernel(xf, scale_vec)
    hist = jnp.sum(part_hist, axis=0)
    edges = jnp.linspace(0.0, x_max, NUM_BINS + 1).astype(jnp.float32)
    return (x, hist, edges)
